# baseline (speedup 1.0000x reference)
import jax
import jax.numpy as jnp
from jax.experimental import pallas as pl

_GRID = 4


def _pe_kernel(row_ref, col_ref, out_ref):
    hb, w, c = out_ref.shape
    d2 = c // 2
    rows = row_ref[...]
    cols = col_ref[0:w, :]
    out_ref[:, :, 0:d2] = jnp.broadcast_to(rows[:, None, :], (hb, w, d2))
    out_ref[:, :, d2:c] = jnp.broadcast_to(cols[None, :, :], (hb, w, d2))


def kernel(x, row_weight, col_weight):
    b, c, h, w = x.shape
    d2 = row_weight.shape[1]
    hb = h // _GRID
    out = pl.pallas_call(
        _pe_kernel,
        grid=(_GRID,),
        in_specs=[
            pl.BlockSpec((hb, d2), lambda i: (i, 0)),
            pl.BlockSpec((w, d2), lambda i: (0, 0)),
        ],
        out_specs=pl.BlockSpec((hb, w, 2 * d2), lambda i: (i, 0, 0)),
        out_shape=jax.ShapeDtypeStruct((h, w, 2 * d2), row_weight.dtype),
    )(row_weight, col_weight)
    return jnp.transpose(out.reshape(1, h, w, 2 * d2), (0, 3, 1, 2))
